# Initial kernel scaffold; baseline (speedup 1.0000x reference)
import jax
import jax.numpy as jnp
from jax.experimental import pallas as pl


def kernel(x):
    raise NotImplementedError("write your pallas kernel here")


# speedup vs baseline: 246.2259x; 246.2259x over previous
import jax
import jax.numpy as jnp
from jax import lax
from jax.experimental import pallas as pl
from jax.experimental.pallas import tpu as pltpu

_K = 10000


def _bisect_mask_kernel(x_ref, o_ref):
    bits = lax.bitcast_convert_type(x_ref[...], jnp.int32)
    sgn = lax.shift_right_arithmetic(bits, 31)
    ukey = lax.bitcast_convert_type(
        bits ^ (sgn | jnp.int32(-2147483648)), jnp.uint32
    )

    def body(i, prefix):
        bit = lax.shift_right_logical(
            jnp.uint32(0x80000000), i.astype(jnp.uint32)
        )
        cand = prefix | bit
        cnt = jnp.sum((ukey >= cand).astype(jnp.int32))
        return jnp.where(cnt >= _K, cand, prefix)

    kth = lax.fori_loop(0, 32, body, jnp.uint32(0))
    o_ref[...] = (ukey >= kth).astype(jnp.float32)


def kernel(x):
    return pl.pallas_call(
        _bisect_mask_kernel,
        out_shape=jax.ShapeDtypeStruct(x.shape, jnp.float32),
    )(x)
